# baseline (speedup 1.0000x reference)
import functools

import jax
import jax.numpy as jnp
from jax import lax
from jax.experimental import pallas as pl
from jax.experimental.pallas import tpu as pltpu
from jax.experimental.pallas import tpu_sc as plsc

B, L, D = 4096, 200, 128
NC, NS = 2, 16
NW = NC * NS
ROWS_PER_W = B // NW
CHUNK = 50
CHUNKS_PER_ROW = L // CHUNK
CHUNKS_PER_W = ROWS_PER_W * CHUNKS_PER_ROW
NLANE = 16
NVEC = D // NLANE

ROWS_PER_GROUP = 2
NBUF = CHUNKS_PER_ROW * ROWS_PER_GROUP
GROUPS = CHUNKS_PER_W // NBUF

_MESH = plsc.VectorSubcoreMesh(
    core_axis_name="c", subcore_axis_name="s", num_cores=NC, num_subcores=NS
)


def _rsqrt16(sv):
    i = plsc.bitcast(sv, jnp.int32)
    y = plsc.bitcast(jnp.int32(0x5F3759DF) - (i >> 1), jnp.float32)
    for _ in range(3):
        y = y * (1.5 - 0.5 * sv * y * y)
    return y


@functools.partial(
    pl.kernel,
    out_type=jax.ShapeDtypeStruct((B, D), jnp.float32),
    mesh=_MESH,
    scratch_types=(
        [pltpu.VMEM((CHUNKS_PER_W // 2, 2, CHUNK), jnp.int32)]
        + [pltpu.VMEM((CHUNK, D), jnp.float32)] * NBUF
        + [pltpu.VMEM((ROWS_PER_GROUP, D), jnp.float32)]
        + [pltpu.SemaphoreType.DMA] * (NBUF + 1)
    ),
    compiler_params=pltpu.CompilerParams(needs_layout_passes=False),
)
def _sc_embed_pool(table_hbm, idx_hbm, out_hbm, idx_v, *bufs):
    rows = bufs[:NBUF]
    out_v = bufs[NBUF]
    sems = bufs[NBUF + 1:NBUF + 1 + NBUF]
    outsem = bufs[NBUF + 1 + NBUF]
    wid = lax.axis_index("s") * NC + lax.axis_index("c")
    cbase = wid * (CHUNKS_PER_W // 2)
    pltpu.sync_copy(idx_hbm.at[pl.ds(cbase, CHUNKS_PER_W // 2)], idx_v)

    for b in range(NBUF):
        pltpu.async_copy(
            table_hbm.at[idx_v.at[b // 2, b % 2]], rows[b], sems[b]
        )

    def group_fn(g, carry):
        @pl.when(g > 0)
        def _drain_out():
            pltpu.make_async_copy(
                out_v, out_hbm.at[pl.ds(0, ROWS_PER_GROUP)], outsem
            ).wait()

        for rr in range(ROWS_PER_GROUP):
            accs = tuple(jnp.zeros((NLANE,), jnp.float32) for _ in range(NVEC))
            for h in range(CHUNKS_PER_ROW):
                k = CHUNKS_PER_ROW * rr + h

                pltpu.make_async_copy(
                    table_hbm.at[idx_v.at[0, 0]], rows[k], sems[k]
                ).wait()

                def j_fn(j, a, _rv=rows[k]):
                    return tuple(
                        a[d] + _rv[j, pl.ds(NLANE * d, NLANE)]
                        for d in range(NVEC)
                    )

                accs = lax.fori_loop(0, CHUNK, j_fn, accs, unroll=4)

                nxt = NBUF * g + k + NBUF

                @pl.when(nxt < CHUNKS_PER_W)
                def _refill(_k=k, _nxt=nxt):
                    pltpu.async_copy(
                        table_hbm.at[idx_v.at[_nxt // 2, _k % 2]],
                        rows[_k],
                        sems[_k],
                    )

            ssq = accs[0] * accs[0]
            for d in range(1, NVEC):
                ssq = ssq + accs[d] * accs[d]
            s = ssq[0]
            for lane in range(1, NLANE):
                s = s + ssq[lane]
            s = s * jnp.float32((1.0 / L) ** 2)
            sv = jnp.maximum(jnp.broadcast_to(s, (NLANE,)), jnp.float32(1e-12))
            scale = _rsqrt16(sv) * jnp.float32(1.0 / L)
            for d in range(NVEC):
                out_v[rr, pl.ds(NLANE * d, NLANE)] = accs[d] * scale

        pltpu.async_copy(
            out_v,
            out_hbm.at[pl.ds(wid * ROWS_PER_W + ROWS_PER_GROUP * g,
                             ROWS_PER_GROUP)],
            outsem,
        )
        return carry

    lax.fori_loop(0, GROUPS, group_fn, 0)
    pltpu.make_async_copy(
        out_v, out_hbm.at[pl.ds(0, ROWS_PER_GROUP)], outsem
    ).wait()


def kernel(indices, emb_table):
    idx2 = indices.reshape(B * CHUNKS_PER_ROW // 2, 2, CHUNK).astype(jnp.int32)
    return _sc_embed_pool(emb_table, idx2)


# speedup vs baseline: 1.0337x; 1.0337x over previous
import functools

import jax
import jax.numpy as jnp
from jax import lax
from jax.experimental import pallas as pl
from jax.experimental.pallas import tpu as pltpu
from jax.experimental.pallas import tpu_sc as plsc

B, L, D = 4096, 200, 128
NC, NS = 2, 16
NW = NC * NS
ROWS_PER_W = B // NW
CHUNK = 100
CHUNKS_PER_ROW = L // CHUNK
CHUNKS_PER_W = ROWS_PER_W * CHUNKS_PER_ROW
NLANE = 16
NVEC = D // NLANE

ROWS_PER_GROUP = 2
NBUF = CHUNKS_PER_ROW * ROWS_PER_GROUP
GROUPS = CHUNKS_PER_W // NBUF

_MESH = plsc.VectorSubcoreMesh(
    core_axis_name="c", subcore_axis_name="s", num_cores=NC, num_subcores=NS
)


def _rsqrt16(sv):
    i = plsc.bitcast(sv, jnp.int32)
    y = plsc.bitcast(jnp.int32(0x5F3759DF) - (i >> 1), jnp.float32)
    for _ in range(3):
        y = y * (1.5 - 0.5 * sv * y * y)
    return y


@functools.partial(
    pl.kernel,
    out_type=jax.ShapeDtypeStruct((B, D), jnp.float32),
    mesh=_MESH,
    scratch_types=(
        [pltpu.VMEM((CHUNKS_PER_W // 2, 2, CHUNK), jnp.int32)]
        + [pltpu.VMEM((CHUNK, D), jnp.float32)] * NBUF
        + [pltpu.VMEM((ROWS_PER_GROUP, D), jnp.float32)]
        + [pltpu.SemaphoreType.DMA] * (NBUF + 1)
    ),
    compiler_params=pltpu.CompilerParams(needs_layout_passes=False),
)
def _sc_embed_pool(table_hbm, idx_hbm, out_hbm, idx_v, *bufs):
    rows = bufs[:NBUF]
    out_v = bufs[NBUF]
    sems = bufs[NBUF + 1:NBUF + 1 + NBUF]
    outsem = bufs[NBUF + 1 + NBUF]
    wid = lax.axis_index("s") * NC + lax.axis_index("c")
    cbase = wid * (CHUNKS_PER_W // 2)
    pltpu.sync_copy(idx_hbm.at[pl.ds(cbase, CHUNKS_PER_W // 2)], idx_v)

    for b in range(NBUF):
        pltpu.async_copy(
            table_hbm.at[idx_v.at[b // 2, b % 2]], rows[b], sems[b]
        )

    def group_fn(g, carry):
        @pl.when(g > 0)
        def _drain_out():
            pltpu.make_async_copy(
                out_v, out_hbm.at[pl.ds(0, ROWS_PER_GROUP)], outsem
            ).wait()

        for rr in range(ROWS_PER_GROUP):
            accs = tuple(jnp.zeros((NLANE,), jnp.float32) for _ in range(NVEC))
            for h in range(CHUNKS_PER_ROW):
                k = CHUNKS_PER_ROW * rr + h

                pltpu.make_async_copy(
                    table_hbm.at[idx_v.at[0, 0]], rows[k], sems[k]
                ).wait()

                def j_fn(j, a, _rv=rows[k]):
                    return tuple(
                        a[d] + _rv[j, pl.ds(NLANE * d, NLANE)]
                        for d in range(NVEC)
                    )

                accs = lax.fori_loop(0, CHUNK, j_fn, accs, unroll=4)

                nxt = NBUF * g + k + NBUF

                @pl.when(nxt < CHUNKS_PER_W)
                def _refill(_k=k, _nxt=nxt):
                    pltpu.async_copy(
                        table_hbm.at[idx_v.at[_nxt // 2, _k % 2]],
                        rows[_k],
                        sems[_k],
                    )

            ssq = accs[0] * accs[0]
            for d in range(1, NVEC):
                ssq = ssq + accs[d] * accs[d]
            s = ssq[0]
            for lane in range(1, NLANE):
                s = s + ssq[lane]
            s = s * jnp.float32((1.0 / L) ** 2)
            sv = jnp.maximum(jnp.broadcast_to(s, (NLANE,)), jnp.float32(1e-12))
            scale = _rsqrt16(sv) * jnp.float32(1.0 / L)
            for d in range(NVEC):
                out_v[rr, pl.ds(NLANE * d, NLANE)] = accs[d] * scale

        pltpu.async_copy(
            out_v,
            out_hbm.at[pl.ds(wid * ROWS_PER_W + ROWS_PER_GROUP * g,
                             ROWS_PER_GROUP)],
            outsem,
        )
        return carry

    lax.fori_loop(0, GROUPS, group_fn, 0)
    pltpu.make_async_copy(
        out_v, out_hbm.at[pl.ds(0, ROWS_PER_GROUP)], outsem
    ).wait()


def kernel(indices, emb_table):
    idx2 = indices.reshape(B * CHUNKS_PER_ROW // 2, 2, CHUNK).astype(jnp.int32)
    return _sc_embed_pool(emb_table, idx2)
